# Initial kernel scaffold; baseline (speedup 1.0000x reference)
import jax
import jax.numpy as jnp
from jax.experimental import pallas as pl


def kernel(x, c, edge_index, W_x, b_x, W_xx, b_xx, W_c, b_c, W_pool, b_pool):
    raise NotImplementedError("write your pallas kernel here")


# speedup vs baseline: 15.7294x; 15.7294x over previous
import functools

import jax
import jax.numpy as jnp
from jax import lax
from jax.experimental import pallas as pl
from jax.experimental.pallas import tpu as pltpu
from jax.experimental.pallas import tpu_sc as plsc

CHUNK = 128
NW = 32
ROW_BLOCK = 1000


def _pre_body(x_ref, wx_ref, bx_ref, wxx_ref, bxx_ref, c_ref, wc_ref,
              bc_ref, self_ref, hxx_ref, cnew_ref):
    xb = x_ref[...]
    self_ref[...] = jnp.maximum(
        jnp.dot(xb, wx_ref[...], preferred_element_type=jnp.float32)
        + bx_ref[...], 0.0)
    hxx_ref[...] = jnp.maximum(
        jnp.dot(xb, wxx_ref[...], preferred_element_type=jnp.float32)
        + bxx_ref[...], 0.0)
    cnew_ref[...] = jnp.maximum(
        jnp.dot(c_ref[...], wc_ref[...], preferred_element_type=jnp.float32)
        + bc_ref[...], 0.0)


def _tc_pre(x, W_x, b_x, W_xx, b_xx, c, W_c, b_c):
    n, in_f = x.shape
    hid = W_x.shape[1]
    ctx = c.shape[1]
    grid = (n // ROW_BLOCK,)
    return pl.pallas_call(
        _pre_body,
        grid=grid,
        in_specs=[
            pl.BlockSpec((ROW_BLOCK, in_f), lambda i: (i, 0)),
            pl.BlockSpec((in_f, hid), lambda i: (0, 0)),
            pl.BlockSpec((1, hid), lambda i: (0, 0)),
            pl.BlockSpec((in_f, hid), lambda i: (0, 0)),
            pl.BlockSpec((1, hid), lambda i: (0, 0)),
            pl.BlockSpec((1, ctx), lambda i: (0, 0)),
            pl.BlockSpec((ctx, hid), lambda i: (0, 0)),
            pl.BlockSpec((1, hid), lambda i: (0, 0)),
        ],
        out_specs=[
            pl.BlockSpec((ROW_BLOCK, hid), lambda i: (i, 0)),
            pl.BlockSpec((ROW_BLOCK, hid), lambda i: (i, 0)),
            pl.BlockSpec((1, hid), lambda i: (0, 0)),
        ],
        out_shape=[
            jax.ShapeDtypeStruct((n, hid), jnp.float32),
            jax.ShapeDtypeStruct((n, hid), jnp.float32),
            jax.ShapeDtypeStruct((1, hid), jnp.float32),
        ],
    )(x, W_x, b_x.reshape(1, -1), W_xx, b_xx.reshape(1, -1),
      c, W_c, b_c.reshape(1, -1))


def _sc_aggregate(hxx, src3, dst3, zrows, n_pad, cpt):
    hid = hxx.shape[1]
    rps = n_pad // 16
    mesh = plsc.VectorSubcoreMesh(core_axis_name="c", subcore_axis_name="s")

    @functools.partial(
        pl.kernel,
        mesh=mesh,
        out_type=jax.ShapeDtypeStruct((2 * n_pad, hid), jnp.float32),
        scratch_types=[
            pltpu.VMEM((cpt, CHUNK), jnp.int32),
            pltpu.VMEM((cpt, CHUNK), jnp.int32),
            pltpu.VMEM((CHUNK, hid), jnp.float32),
            pltpu.VMEM_SHARED((n_pad, hid), jnp.float32),
            pltpu.SemaphoreType.DMA,
        ],
        compiler_params=pltpu.CompilerParams(use_tc_tiling_on_sc=False),
    )
    def k(hxx_hbm, src_hbm, dst_hbm, z_hbm, out_hbm,
          src_v, dst_v, rows_v, acc_sh, sem):
        cid = lax.axis_index("c")
        sid = lax.axis_index("s")
        wid = cid * 16 + sid
        pltpu.sync_copy(z_hbm, acc_sh.at[pl.ds(sid * rps, rps)])
        pltpu.sync_copy(src_hbm.at[wid], src_v)
        pltpu.sync_copy(dst_hbm.at[wid], dst_v)
        plsc.subcore_barrier()

        def body(j, carry):
            pltpu.async_copy(hxx_hbm.at[src_v.at[j]], rows_v, sem).wait()
            pltpu.sync_copy(rows_v, acc_sh.at[dst_v.at[j]], add=True)
            return carry

        lax.fori_loop(0, cpt, body, 0)
        plsc.subcore_barrier()
        pltpu.sync_copy(acc_sh.at[pl.ds(sid * rps, rps)],
                        out_hbm.at[pl.ds(cid * n_pad + sid * rps, rps)])

    return k(hxx, src3, dst3, zrows)


def _post_body(p0_ref, p1_ref, self_ref, wp_ref, bp_ref, out_ref):
    aggr = p0_ref[...] + p1_ref[...] + self_ref[...]
    out_ref[...] = (
        jnp.dot(aggr, wp_ref[...], preferred_element_type=jnp.float32)
        + bp_ref[...])


def _tc_post(p0, p1, self_x, W_pool, b_pool):
    n, hid = self_x.shape
    out_f = W_pool.shape[1]
    grid = (n // ROW_BLOCK,)
    return pl.pallas_call(
        _post_body,
        grid=grid,
        in_specs=[
            pl.BlockSpec((ROW_BLOCK, hid), lambda i: (i, 0)),
            pl.BlockSpec((ROW_BLOCK, hid), lambda i: (i, 0)),
            pl.BlockSpec((ROW_BLOCK, hid), lambda i: (i, 0)),
            pl.BlockSpec((hid, out_f), lambda i: (0, 0)),
            pl.BlockSpec((1, out_f), lambda i: (0, 0)),
        ],
        out_specs=pl.BlockSpec((ROW_BLOCK, out_f), lambda i: (i, 0)),
        out_shape=jax.ShapeDtypeStruct((n, out_f), jnp.float32),
    )(p0, p1, self_x, W_pool, b_pool.reshape(1, -1))


def kernel(x, c, edge_index, W_x, b_x, W_xx, b_xx, W_c, b_c, W_pool, b_pool):
    n = x.shape[0]
    hid = W_x.shape[1]
    ei = edge_index.astype(jnp.int32)
    src, dst = ei[0], ei[1]
    e = src.shape[0]

    cpt = -(-e // (NW * CHUNK))
    e_pad = NW * cpt * CHUNK
    n_pad = ((n // 128) + 1) * 128
    rps = n_pad // 16

    src_p = jnp.concatenate(
        [src, jnp.zeros((e_pad - e,), jnp.int32)]).reshape(NW, cpt, CHUNK)
    dst_p = jnp.concatenate(
        [dst, jnp.full((e_pad - e,), n, jnp.int32)]).reshape(NW, cpt, CHUNK)
    zrows = jnp.zeros((rps, hid), jnp.float32)

    self_x, h_xx, c_new = _tc_pre(x, W_x, b_x, W_xx, b_xx, c, W_c, b_c)
    partials = _sc_aggregate(h_xx, src_p, dst_p, zrows, n_pad, cpt)
    p0 = partials[:n]
    p1 = partials[n_pad:n_pad + n]
    x_new = _tc_post(p0, p1, self_x, W_pool, b_pool)
    return (x_new, c_new)
